# baseline (speedup 1.0000x reference)
import functools

import jax
import jax.numpy as jnp
from jax import lax
from jax.experimental import pallas as pl
from jax.experimental.pallas import tpu as pltpu
from jax.experimental.pallas import tpu_sc as plsc

_B, _L = 16, 4096
_PAGE = 64
_P = _L // _PAGE
_N = _B * _L
_NPAGES = _B * _P
_NC, _NS, _LANES = 2, 16, 16
_NW = _NC * _NS
_TOK_W = _N // _NW
_PG_W = _NPAGES // _NW
_MEAN_W, _MAX_W = 0.7, 0.3
_NEG = -1e9


def _seg_body(mask_hbm, score_hbm, t2p_hbm, ps_hbm, pv_hbm,
              mask_v, score_v, t2p_v, ps_v, pv_v):
    wid = lax.axis_index("s") * _NC + lax.axis_index("c")
    base = wid * _TOK_W
    pltpu.sync_copy(mask_hbm.at[pl.ds(base, _TOK_W)], mask_v)
    pltpu.sync_copy(score_hbm.at[pl.ds(base, _TOK_W)], score_v)

    page0 = (wid % (_L // _TOK_W)) * _PG_W
    lane = lax.iota(jnp.int32, _LANES)
    neg1 = jnp.full((_LANES,), -1, jnp.int32)
    zero_f = jnp.zeros((_LANES,), jnp.float32)
    one_i = jnp.full((_LANES,), 1, jnp.int32)
    zero_i = jnp.zeros((_LANES,), jnp.int32)

    for g in range(_PG_W // _LANES):
        base_idx = (lane + g * _LANES) * _PAGE
        page_vec = lane + (page0 + g * _LANES)

        def body(j, carry, base_idx=base_idx, page_vec=page_vec):
            s, mx, cnt = carry
            idx = base_idx + j
            sc = plsc.load_gather(score_v, [idx])
            mk = plsc.load_gather(mask_v, [idx])
            valid = mk != 0
            plsc.store_scatter(t2p_v, [idx], jnp.where(valid, page_vec, neg1))
            s = s + jnp.where(valid, sc, zero_f)
            mx = jnp.maximum(mx, jnp.where(valid, sc, _NEG))
            cnt = cnt + jnp.where(valid, one_i, zero_i)
            return (s, mx, cnt)

        s0 = jnp.zeros((_LANES,), jnp.float32)
        mx0 = jnp.full((_LANES,), _NEG, jnp.float32)
        c0 = jnp.zeros((_LANES,), jnp.int32)
        s, mx, cnt = lax.fori_loop(0, _PAGE, body, (s0, mx0, c0), unroll=8)

        cntf = jnp.maximum(cnt, 1).astype(jnp.float32)
        raw = _MEAN_W * (s / cntf) + _MAX_W * mx
        valid_page = cnt > 0
        ps_v[pl.ds(g * _LANES, _LANES)] = jnp.where(valid_page, raw, zero_f)
        pv_v[pl.ds(g * _LANES, _LANES)] = jnp.where(valid_page, one_i, zero_i)

    pltpu.sync_copy(t2p_v, t2p_hbm.at[pl.ds(base, _TOK_W)])
    pltpu.sync_copy(ps_v, ps_hbm.at[pl.ds(wid * _PG_W, _PG_W)])
    pltpu.sync_copy(pv_v, pv_hbm.at[pl.ds(wid * _PG_W, _PG_W)])


@functools.lru_cache(maxsize=1)
def _build_seg_kernel():
    return functools.partial(
        pl.kernel,
        out_type=(
            jax.ShapeDtypeStruct((_N,), jnp.int32),
            jax.ShapeDtypeStruct((_NPAGES,), jnp.float32),
            jax.ShapeDtypeStruct((_NPAGES,), jnp.int32),
        ),
        mesh=plsc.VectorSubcoreMesh(core_axis_name="c", subcore_axis_name="s"),
        compiler_params=pltpu.CompilerParams(
            needs_layout_passes=False,
            disable_bounds_checks=True,
            disable_semaphore_checks=True,
        ),
        scratch_types=[
            pltpu.VMEM((_TOK_W,), jnp.int32),
            pltpu.VMEM((_TOK_W,), jnp.float32),
            pltpu.VMEM((_TOK_W,), jnp.int32),
            pltpu.VMEM((_PG_W,), jnp.float32),
            pltpu.VMEM((_PG_W,), jnp.int32),
        ],
    )(_seg_body)


def kernel(input_ids, attention_mask, token_scores):
    del input_ids
    mask_flat = attention_mask.reshape(_N)
    score_flat = token_scores.reshape(_N)
    t2p, ps, pv = _build_seg_kernel()(mask_flat, score_flat)
    return (ps.reshape(_B, _P),
            t2p.reshape(_B, _L),
            pv.reshape(_B, _P).astype(bool))


# speedup vs baseline: 1.1120x; 1.1120x over previous
import functools

import jax
import jax.numpy as jnp
from jax import lax
from jax.experimental import pallas as pl
from jax.experimental.pallas import tpu as pltpu
from jax.experimental.pallas import tpu_sc as plsc

_B, _L = 16, 4096
_PAGE = 64
_P = _L // _PAGE
_N = _B * _L
_NPAGES = _B * _P
_NC, _NS, _LANES = 2, 16, 16
_NW = _NC * _NS
_TOK_W = _N // _NW
_PG_W = _NPAGES // _NW
_MEAN_W, _MAX_W = 0.7, 0.3
_NEG = -1e9


def _seg_body(mask_hbm, score_hbm, t2p_hbm, ps_hbm, pv_hbm,
              mask_v, score_v, t2p_v, ps_v, pv_v):
    wid = lax.axis_index("s") * _NC + lax.axis_index("c")
    base = wid * _TOK_W
    pltpu.sync_copy(mask_hbm.at[pl.ds(base, _TOK_W)], mask_v)
    pltpu.sync_copy(score_hbm.at[pl.ds(base, _TOK_W)], score_v)

    page0 = (wid % (_L // _TOK_W)) * _PG_W
    lane = lax.iota(jnp.int32, _LANES)
    neg1 = jnp.full((_LANES,), -1, jnp.int32)
    zero_f = jnp.zeros((_LANES,), jnp.float32)
    one_i = jnp.full((_LANES,), 1, jnp.int32)
    zero_i = jnp.zeros((_LANES,), jnp.int32)
    m63 = jnp.full((_LANES,), _PAGE - 1, jnp.int32)

    for g in range(_PG_W // _LANES):
        page_base = lane * _PAGE + g * _LANES * _PAGE
        page_vec = lane + (page0 + g * _LANES)

        def body(j, carry, page_base=page_base, page_vec=page_vec):
            s, mx, cnt = carry
            idx = page_base + ((lane + j) & m63)
            sc = plsc.load_gather(score_v, [idx])
            mk = plsc.load_gather(mask_v, [idx])
            valid = mk != 0
            plsc.store_scatter(t2p_v, [idx], jnp.where(valid, page_vec, neg1))
            s = s + jnp.where(valid, sc, zero_f)
            mx = jnp.maximum(mx, jnp.where(valid, sc, _NEG))
            cnt = cnt + jnp.where(valid, one_i, zero_i)
            return (s, mx, cnt)

        s0 = jnp.zeros((_LANES,), jnp.float32)
        mx0 = jnp.full((_LANES,), _NEG, jnp.float32)
        c0 = jnp.zeros((_LANES,), jnp.int32)
        s, mx, cnt = lax.fori_loop(0, _PAGE, body, (s0, mx0, c0), unroll=8)

        cntf = jnp.maximum(cnt, 1).astype(jnp.float32)
        raw = _MEAN_W * (s / cntf) + _MAX_W * mx
        valid_page = cnt > 0
        ps_v[pl.ds(g * _LANES, _LANES)] = jnp.where(valid_page, raw, zero_f)
        pv_v[pl.ds(g * _LANES, _LANES)] = jnp.where(valid_page, one_i, zero_i)

    pltpu.sync_copy(t2p_v, t2p_hbm.at[pl.ds(base, _TOK_W)])
    pltpu.sync_copy(ps_v, ps_hbm.at[pl.ds(wid * _PG_W, _PG_W)])
    pltpu.sync_copy(pv_v, pv_hbm.at[pl.ds(wid * _PG_W, _PG_W)])


@functools.lru_cache(maxsize=1)
def _build_seg_kernel():
    return functools.partial(
        pl.kernel,
        out_type=(
            jax.ShapeDtypeStruct((_N,), jnp.int32),
            jax.ShapeDtypeStruct((_NPAGES,), jnp.float32),
            jax.ShapeDtypeStruct((_NPAGES,), jnp.int32),
        ),
        mesh=plsc.VectorSubcoreMesh(core_axis_name="c", subcore_axis_name="s"),
        compiler_params=pltpu.CompilerParams(
            needs_layout_passes=False,
            disable_bounds_checks=True,
            disable_semaphore_checks=True,
        ),
        scratch_types=[
            pltpu.VMEM((_TOK_W,), jnp.int32),
            pltpu.VMEM((_TOK_W,), jnp.float32),
            pltpu.VMEM((_TOK_W,), jnp.int32),
            pltpu.VMEM((_PG_W,), jnp.float32),
            pltpu.VMEM((_PG_W,), jnp.int32),
        ],
    )(_seg_body)


def kernel(input_ids, attention_mask, token_scores):
    del input_ids
    mask_flat = attention_mask.reshape(_N)
    score_flat = token_scores.reshape(_N)
    t2p, ps, pv = _build_seg_kernel()(mask_flat, score_flat)
    return (ps.reshape(_B, _P),
            t2p.reshape(_B, _L),
            pv.reshape(_B, _P).astype(bool))
